# baseline (speedup 1.0000x reference)
import functools

import jax
import jax.numpy as jnp
import numpy as np
from jax import lax
from jax.experimental import pallas as pl
from jax.experimental.pallas import tpu as pltpu
from jax.experimental.pallas import tpu_sc as plsc

VOCAB = 1000000
EMBED_DIM = 32
SEQ = 50
BATCH = 4096
NC, NS, LANES = 2, 16, 16
NW = NC * NS
SLABS = 2 * BATCH
SLABS_PER_CHUNK = 2
TOK_PER_CHUNK = SLABS_PER_CHUNK * SEQ
CHUNK_FLOATS = TOK_PER_CHUNK * EMBED_DIM
CHUNKS_PER_W = SLABS // (NW * SLABS_PER_CHUNK)
NBUF = 4


def _make_gather_idx() -> np.ndarray:
    o = np.arange(2 * TOK_PER_CHUNK * LANES, dtype=np.int32)
    s, r = np.divmod(o, EMBED_DIM * SEQ)
    d, l = np.divmod(r, SEQ)
    trow = (s * SEQ + l).reshape(2 * TOK_PER_CHUNK, LANES)
    tcol = d.reshape(2 * TOK_PER_CHUNK, LANES)
    return np.stack([trow, tcol]).astype(np.int32)


_TIDX = _make_gather_idx()


def _body(table_hbm, idx_hbm, tidx_hbm, out_hbm, idx_v, tidx_v, rows_v,
          obuf_v, gsems, wsems):
    c = lax.axis_index("c")
    s = lax.axis_index("s")
    w = s * NC + c
    pltpu.sync_copy(idx_hbm.at[w], idx_v)
    pltpu.sync_copy(tidx_hbm, tidx_v)

    nbuf = len(gsems)
    for b in range(nbuf):
        pltpu.async_copy(table_hbm.at[idx_v.at[b]], rows_v.at[b], gsems[b])

    @pl.loop(0, CHUNKS_PER_W // nbuf)
    def chunk(g):
        for b in range(nbuf):
            cg = g * nbuf + b
            pltpu.make_async_copy(table_hbm.at[idx_v.at[cg]], rows_v.at[b],
                                  gsems[b]).wait()

            @pl.when(g >= 1)
            def _():
                pltpu.make_async_copy(obuf_v.at[b], out_hbm.at[w, cg - nbuf],
                                      wsems[b]).wait()

            for m in range(2 * TOK_PER_CHUNK):
                v = plsc.load_gather(rows_v.at[b],
                                     [tidx_v[0, m], tidx_v[1, m]])
                obuf_v[b, pl.ds(m * LANES, LANES)] = v
            pltpu.async_copy(obuf_v.at[b], out_hbm.at[w, cg], wsems[b])

            @pl.when(cg + nbuf < CHUNKS_PER_W)
            def _():
                pltpu.async_copy(table_hbm.at[idx_v.at[cg + nbuf]],
                                 rows_v.at[b], gsems[b])

    for b in range(nbuf):
        pltpu.make_async_copy(obuf_v.at[b],
                              out_hbm.at[w, CHUNKS_PER_W - nbuf + b],
                              wsems[b]).wait()


@functools.partial(jax.jit, donate_argnums=())
def _run(table, idx3, tidx):
    mesh = plsc.VectorSubcoreMesh(core_axis_name="c", subcore_axis_name="s",
                                  num_cores=NC, num_subcores=NS)
    kern = pl.kernel(
        _body,
        out_type=jax.ShapeDtypeStruct((NW, CHUNKS_PER_W, CHUNK_FLOATS),
                                      jnp.float32),
        mesh=mesh,
        scratch_types=[
            pltpu.VMEM((CHUNKS_PER_W, TOK_PER_CHUNK), jnp.int32),
            pltpu.VMEM((2, 2 * TOK_PER_CHUNK, LANES), jnp.int32),
            pltpu.VMEM((NBUF, TOK_PER_CHUNK, EMBED_DIM), jnp.float32),
            pltpu.VMEM((NBUF, CHUNK_FLOATS), jnp.float32),
            [pltpu.SemaphoreType.DMA] * NBUF,
            [pltpu.SemaphoreType.DMA] * NBUF,
        ],
        compiler_params=pltpu.CompilerParams(needs_layout_passes=False,
                                             use_tc_tiling_on_sc=False),
    )
    return kern(table, idx3, tidx)


def kernel(query_input, document_input, table):
    idx = jnp.stack([query_input, document_input]).astype(jnp.int32)
    idx3 = idx.reshape(NW, CHUNKS_PER_W, TOK_PER_CHUNK)
    tidx = jnp.asarray(_TIDX)
    out = _run(table, idx3, tidx)
    return out.reshape(2, BATCH, EMBED_DIM, SEQ)


# speedup vs baseline: 1.1690x; 1.1690x over previous
import functools

import jax
import jax.numpy as jnp
import numpy as np
from jax import lax
from jax.experimental import pallas as pl
from jax.experimental.pallas import tpu as pltpu
from jax.experimental.pallas import tpu_sc as plsc

VOCAB = 1000000
EMBED_DIM = 32
SEQ = 50
BATCH = 4096
NC, NS, LANES = 2, 16, 16
NW = NC * NS
SLABS = 2 * BATCH
SLABS_PER_CHUNK = 2
TOK_PER_CHUNK = SLABS_PER_CHUNK * SEQ
CHUNK_FLOATS = TOK_PER_CHUNK * EMBED_DIM
CHUNKS_PER_W = SLABS // (NW * SLABS_PER_CHUNK)
NBUF = 4


def _make_scatter_idx() -> np.ndarray:
    sidx = np.empty((2 * TOK_PER_CHUNK, LANES), np.int32)
    k = np.arange(LANES, dtype=np.int32)
    for j in range(TOK_PER_CHUNK):
        s, l = divmod(j, SEQ)
        for h in range(2):
            sidx[2 * j + h] = s * (EMBED_DIM * SEQ) + l + SEQ * (h * LANES + k)
    return sidx


_TIDX = _make_scatter_idx()


def _body(table_hbm, idx_hbm, tidx_hbm, out_hbm, idx_v, tidx_v, rows_v,
          obuf_v, gsems, wsems):
    c = lax.axis_index("c")
    s = lax.axis_index("s")
    w = s * NC + c
    pltpu.sync_copy(idx_hbm.at[w], idx_v)
    pltpu.sync_copy(tidx_hbm, tidx_v)

    nbuf = len(gsems)
    for b in range(nbuf):
        pltpu.async_copy(table_hbm.at[idx_v.at[b]], rows_v.at[b], gsems[b])

    @pl.loop(0, CHUNKS_PER_W // nbuf)
    def chunk(g):
        for b in range(nbuf):
            cg = g * nbuf + b
            pltpu.make_async_copy(table_hbm.at[idx_v.at[cg]], rows_v.at[b],
                                  gsems[b]).wait()

            @pl.when(g >= 1)
            def _():
                pltpu.make_async_copy(obuf_v.at[b], out_hbm.at[w, cg - nbuf],
                                      wsems[b]).wait()

            for j in range(TOK_PER_CHUNK):
                v0 = rows_v[b, j, pl.ds(0, LANES)]
                v1 = rows_v[b, j, pl.ds(LANES, LANES)]
                plsc.store_scatter(obuf_v.at[b], [tidx_v[2 * j]], v0)
                plsc.store_scatter(obuf_v.at[b], [tidx_v[2 * j + 1]], v1)
            pltpu.async_copy(obuf_v.at[b], out_hbm.at[w, cg], wsems[b])

            @pl.when(cg + nbuf < CHUNKS_PER_W)
            def _():
                pltpu.async_copy(table_hbm.at[idx_v.at[cg + nbuf]],
                                 rows_v.at[b], gsems[b])

    for b in range(nbuf):
        pltpu.make_async_copy(obuf_v.at[b],
                              out_hbm.at[w, CHUNKS_PER_W - nbuf + b],
                              wsems[b]).wait()


@functools.partial(jax.jit, donate_argnums=())
def _run(table, idx3, tidx):
    mesh = plsc.VectorSubcoreMesh(core_axis_name="c", subcore_axis_name="s",
                                  num_cores=NC, num_subcores=NS)
    kern = pl.kernel(
        _body,
        out_type=jax.ShapeDtypeStruct((NW, CHUNKS_PER_W, CHUNK_FLOATS),
                                      jnp.float32),
        mesh=mesh,
        scratch_types=[
            pltpu.VMEM((CHUNKS_PER_W, TOK_PER_CHUNK), jnp.int32),
            pltpu.VMEM((2 * TOK_PER_CHUNK, LANES), jnp.int32),
            pltpu.VMEM((NBUF, TOK_PER_CHUNK, EMBED_DIM), jnp.float32),
            pltpu.VMEM((NBUF, CHUNK_FLOATS), jnp.float32),
            [pltpu.SemaphoreType.DMA] * NBUF,
            [pltpu.SemaphoreType.DMA] * NBUF,
        ],
        compiler_params=pltpu.CompilerParams(needs_layout_passes=False,
                                             use_tc_tiling_on_sc=False),
    )
    return kern(table, idx3, tidx)


def kernel(query_input, document_input, table):
    idx = jnp.stack([query_input, document_input]).astype(jnp.int32)
    idx3 = idx.reshape(NW, CHUNKS_PER_W, TOK_PER_CHUNK)
    tidx = jnp.asarray(_TIDX)
    out = _run(table, idx3, tidx)
    return out.reshape(2, BATCH, EMBED_DIM, SEQ)


# speedup vs baseline: 1.1727x; 1.0031x over previous
import functools

import jax
import jax.numpy as jnp
import numpy as np
from jax import lax
from jax.experimental import pallas as pl
from jax.experimental.pallas import tpu as pltpu
from jax.experimental.pallas import tpu_sc as plsc

VOCAB = 1000000
EMBED_DIM = 32
SEQ = 50
BATCH = 4096
NC, NS, LANES = 2, 16, 16
NW = NC * NS
SLABS = 2 * BATCH
SLABS_PER_CHUNK = 4
TOK_PER_CHUNK = SLABS_PER_CHUNK * SEQ
CHUNK_FLOATS = TOK_PER_CHUNK * EMBED_DIM
CHUNKS_PER_W = SLABS // (NW * SLABS_PER_CHUNK)
NBUF = 2


def _make_scatter_idx() -> np.ndarray:
    sidx = np.empty((2 * TOK_PER_CHUNK, LANES), np.int32)
    k = np.arange(LANES, dtype=np.int32)
    for j in range(TOK_PER_CHUNK):
        s, l = divmod(j, SEQ)
        for h in range(2):
            sidx[2 * j + h] = s * (EMBED_DIM * SEQ) + l + SEQ * (h * LANES + k)
    return sidx


_TIDX = _make_scatter_idx()


def _body(table_hbm, idx_hbm, tidx_hbm, out_hbm, idx_v, tidx_v, rows_v,
          obuf_v, gsems, wsems):
    c = lax.axis_index("c")
    s = lax.axis_index("s")
    w = s * NC + c
    pltpu.sync_copy(idx_hbm.at[w], idx_v)
    pltpu.sync_copy(tidx_hbm, tidx_v)

    nbuf = len(gsems)
    for b in range(nbuf):
        pltpu.async_copy(table_hbm.at[idx_v.at[b]], rows_v.at[b], gsems[b])

    @pl.loop(0, CHUNKS_PER_W // nbuf)
    def chunk(g):
        for b in range(nbuf):
            cg = g * nbuf + b
            pltpu.make_async_copy(table_hbm.at[idx_v.at[cg]], rows_v.at[b],
                                  gsems[b]).wait()

            @pl.when(g >= 1)
            def _():
                pltpu.make_async_copy(obuf_v.at[b], out_hbm.at[w, cg - nbuf],
                                      wsems[b]).wait()

            for j in range(TOK_PER_CHUNK):
                v0 = rows_v[b, j, pl.ds(0, LANES)]
                v1 = rows_v[b, j, pl.ds(LANES, LANES)]
                plsc.store_scatter(obuf_v.at[b], [tidx_v[2 * j]], v0)
                plsc.store_scatter(obuf_v.at[b], [tidx_v[2 * j + 1]], v1)
            pltpu.async_copy(obuf_v.at[b], out_hbm.at[w, cg], wsems[b])

            @pl.when(cg + nbuf < CHUNKS_PER_W)
            def _():
                pltpu.async_copy(table_hbm.at[idx_v.at[cg + nbuf]],
                                 rows_v.at[b], gsems[b])

    for b in range(nbuf):
        pltpu.make_async_copy(obuf_v.at[b],
                              out_hbm.at[w, CHUNKS_PER_W - nbuf + b],
                              wsems[b]).wait()


@functools.partial(jax.jit, donate_argnums=())
def _run(table, idx3, tidx):
    mesh = plsc.VectorSubcoreMesh(core_axis_name="c", subcore_axis_name="s",
                                  num_cores=NC, num_subcores=NS)
    kern = pl.kernel(
        _body,
        out_type=jax.ShapeDtypeStruct((NW, CHUNKS_PER_W, CHUNK_FLOATS),
                                      jnp.float32),
        mesh=mesh,
        scratch_types=[
            pltpu.VMEM((CHUNKS_PER_W, TOK_PER_CHUNK), jnp.int32),
            pltpu.VMEM((2 * TOK_PER_CHUNK, LANES), jnp.int32),
            pltpu.VMEM((NBUF, TOK_PER_CHUNK, EMBED_DIM), jnp.float32),
            pltpu.VMEM((NBUF, CHUNK_FLOATS), jnp.float32),
            [pltpu.SemaphoreType.DMA] * NBUF,
            [pltpu.SemaphoreType.DMA] * NBUF,
        ],
        compiler_params=pltpu.CompilerParams(needs_layout_passes=False,
                                             use_tc_tiling_on_sc=False),
    )
    return kern(table, idx3, tidx)


def kernel(query_input, document_input, table):
    idx = jnp.stack([query_input, document_input]).astype(jnp.int32)
    idx3 = idx.reshape(NW, CHUNKS_PER_W, TOK_PER_CHUNK)
    tidx = jnp.asarray(_TIDX)
    out = _run(table, idx3, tidx)
    return out.reshape(2, BATCH, EMBED_DIM, SEQ)


# speedup vs baseline: 1.2632x; 1.0772x over previous
import functools

import jax
import jax.numpy as jnp
import numpy as np
from jax import lax
from jax.experimental import pallas as pl
from jax.experimental.pallas import tpu as pltpu
from jax.experimental.pallas import tpu_sc as plsc

VOCAB = 1000000
EMBED_DIM = 32
SEQ = 50
BATCH = 4096
NC, NS, LANES = 2, 16, 16
NW = NC * NS
SLABS = BATCH
SLABS_PER_CHUNK = 4
TOK_PER_CHUNK = SLABS_PER_CHUNK * SEQ
CHUNK_FLOATS = TOK_PER_CHUNK * EMBED_DIM
CHUNKS_PER_W = SLABS // (NW * SLABS_PER_CHUNK)
NBUF = 2


def _make_scatter_idx() -> np.ndarray:
    sidx = np.empty((2 * TOK_PER_CHUNK, LANES), np.int32)
    k = np.arange(LANES, dtype=np.int32)
    for j in range(TOK_PER_CHUNK):
        s, l = divmod(j, SEQ)
        for h in range(2):
            sidx[2 * j + h] = s * (EMBED_DIM * SEQ) + l + SEQ * (h * LANES + k)
    return sidx


_TIDX = _make_scatter_idx()


def _body(table_hbm, idx_hbm, tidx_hbm, out_hbm, idx_v, tidx_v, rows_v,
          obuf_v, gsems, wsems):
    c = lax.axis_index("c")
    s = lax.axis_index("s")
    w = s * NC + c
    pltpu.sync_copy(idx_hbm.at[w], idx_v)
    pltpu.sync_copy(tidx_hbm, tidx_v)

    nbuf = len(gsems)
    for b in range(nbuf):
        pltpu.async_copy(table_hbm.at[idx_v.at[b]], rows_v.at[b], gsems[b])

    @pl.loop(0, CHUNKS_PER_W // nbuf)
    def chunk(g):
        for b in range(nbuf):
            cg = g * nbuf + b
            pltpu.make_async_copy(table_hbm.at[idx_v.at[cg]], rows_v.at[b],
                                  gsems[b]).wait()

            @pl.when(g >= 1)
            def _():
                pltpu.make_async_copy(obuf_v.at[b], out_hbm.at[w, cg - nbuf],
                                      wsems[b]).wait()

            for j in range(TOK_PER_CHUNK):
                v0 = rows_v[b, j, pl.ds(0, LANES)]
                v1 = rows_v[b, j, pl.ds(LANES, LANES)]
                plsc.store_scatter(obuf_v.at[b], [tidx_v[2 * j]], v0)
                plsc.store_scatter(obuf_v.at[b], [tidx_v[2 * j + 1]], v1)
            pltpu.async_copy(obuf_v.at[b], out_hbm.at[w, cg], wsems[b])

            @pl.when(cg + nbuf < CHUNKS_PER_W)
            def _():
                pltpu.async_copy(table_hbm.at[idx_v.at[cg + nbuf]],
                                 rows_v.at[b], gsems[b])

    for b in range(nbuf):
        pltpu.make_async_copy(obuf_v.at[b],
                              out_hbm.at[w, CHUNKS_PER_W - nbuf + b],
                              wsems[b]).wait()


@functools.partial(jax.jit, donate_argnums=())
def _run(table, idx3, tidx):
    mesh = plsc.VectorSubcoreMesh(core_axis_name="c", subcore_axis_name="s",
                                  num_cores=NC, num_subcores=NS)
    kern = pl.kernel(
        _body,
        out_type=jax.ShapeDtypeStruct((NW, CHUNKS_PER_W, CHUNK_FLOATS),
                                      jnp.float32),
        mesh=mesh,
        scratch_types=[
            pltpu.VMEM((CHUNKS_PER_W, TOK_PER_CHUNK), jnp.int32),
            pltpu.VMEM((2 * TOK_PER_CHUNK, LANES), jnp.int32),
            pltpu.VMEM((NBUF, TOK_PER_CHUNK, EMBED_DIM), jnp.float32),
            pltpu.VMEM((NBUF, CHUNK_FLOATS), jnp.float32),
            [pltpu.SemaphoreType.DMA] * NBUF,
            [pltpu.SemaphoreType.DMA] * NBUF,
        ],
        compiler_params=pltpu.CompilerParams(needs_layout_passes=False,
                                             use_tc_tiling_on_sc=False),
    )
    return kern(table, idx3, tidx)


def kernel(query_input, document_input, table):
    tidx = jnp.asarray(_TIDX)
    idx_q = query_input.astype(jnp.int32).reshape(NW, CHUNKS_PER_W,
                                                  TOK_PER_CHUNK)
    idx_d = document_input.astype(jnp.int32).reshape(NW, CHUNKS_PER_W,
                                                     TOK_PER_CHUNK)
    out_q = _run(table, idx_q, tidx).reshape(BATCH, EMBED_DIM, SEQ)
    out_d = _run(table, idx_d, tidx).reshape(BATCH, EMBED_DIM, SEQ)
    return jnp.stack([out_q, out_d])


# speedup vs baseline: 1.7478x; 1.3836x over previous
import functools

import jax
import jax.numpy as jnp
from jax import lax
from jax.experimental import pallas as pl
from jax.experimental.pallas import tpu as pltpu
from jax.experimental.pallas import tpu_sc as plsc

VOCAB = 1000000
EMBED_DIM = 32
SEQ = 50
BATCH = 4096
NC, NS, LANES = 2, 16, 16
NW = NC * NS
BBLK = BATCH // NW
IDXT_PITCH = 136
OB_PITCH = 131
NBUF = 2
NCHUNK = 2 * SEQ


def _body(table_hbm, q_hbm, d_hbm, out_hbm, idxt_v, rows_v, obuf_v, iraw_v,
          gsems, wsems):
    c = lax.axis_index("c")
    s = lax.axis_index("s")
    w = s * NC + c

    iota = lax.iota(jnp.int32, LANES)

    pltpu.sync_copy(q_hbm.at[w], iraw_v.at[0])
    pltpu.sync_copy(d_hbm.at[w], iraw_v.at[1])
    for p in range(2):
        for b in range(BBLK):
            for off in (0, 16, 32, 34):
                v = iraw_v[p, b, pl.ds(off, LANES)]
                dst = (iota + off) * IDXT_PITCH + b
                plsc.store_scatter(idxt_v.at[p], [dst], v)

    for nb in range(NBUF):
        pltpu.async_copy(
            table_hbm.at[idxt_v.at[nb // SEQ, pl.ds((nb % SEQ) * IDXT_PITCH,
                                                    BBLK)]],
            rows_v.at[nb], gsems[nb])

    rt0 = iota // 8
    rt1 = rt0 + 2
    dd0 = iota % 8

    @pl.loop(0, NCHUNK // NBUF)
    def chunk(g):
        for nb in range(NBUF):
            cg = g * NBUF + nb
            p = cg // SEQ
            l = cg - p * SEQ
            pltpu.make_async_copy(
                table_hbm.at[idxt_v.at[p, pl.ds(l * IDXT_PITCH, BBLK)]],
                rows_v.at[nb], gsems[nb]).wait()

            @pl.when(g >= 1)
            def _():
                pg = (cg - NBUF) // SEQ
                lg = (cg - NBUF) - pg * SEQ
                for rt in range(4):
                    pltpu.make_async_copy(
                        obuf_v.at[nb, rt, :, pl.ds(0, BBLK)],
                        out_hbm.at[pg, lg, rt, w], wsems[nb]).wait()

            for bb in range(BBLK):
                v0 = rows_v[nb, bb, pl.ds(0, LANES)]
                v1 = rows_v[nb, bb, pl.ds(LANES, LANES)]
                bbs = jnp.full((LANES,), bb, jnp.int32)
                plsc.store_scatter(obuf_v.at[nb], [rt0, dd0, bbs], v0)
                plsc.store_scatter(obuf_v.at[nb], [rt1, dd0, bbs], v1)
            for rt in range(4):
                pltpu.async_copy(obuf_v.at[nb, rt, :, pl.ds(0, BBLK)],
                                 out_hbm.at[p, l, rt, w], wsems[nb])

            @pl.when(cg + NBUF < NCHUNK)
            def _():
                pn = (cg + NBUF) // SEQ
                ln = (cg + NBUF) - pn * SEQ
                pltpu.async_copy(
                    table_hbm.at[idxt_v.at[pn, pl.ds(ln * IDXT_PITCH, BBLK)]],
                    rows_v.at[nb], gsems[nb])

    for nb in range(NBUF):
        cg = NCHUNK - NBUF + nb
        p = cg // SEQ
        l = cg - p * SEQ
        for rt in range(4):
            pltpu.make_async_copy(obuf_v.at[nb, rt], out_hbm.at[p, l, rt, w],
                                  wsems[nb]).wait()


@functools.partial(jax.jit, donate_argnums=())
def _run(table, q4, d4):
    mesh = plsc.VectorSubcoreMesh(core_axis_name="c", subcore_axis_name="s",
                                  num_cores=NC, num_subcores=NS)
    kern = pl.kernel(
        _body,
        out_type=jax.ShapeDtypeStruct((2, SEQ, 4, NW, 8, BBLK), jnp.float32),
        mesh=mesh,
        scratch_types=[
            pltpu.VMEM((2, SEQ * IDXT_PITCH), jnp.int32),
            pltpu.VMEM((NBUF, BBLK, EMBED_DIM), jnp.float32),
            pltpu.VMEM((NBUF, 4, 8, OB_PITCH), jnp.float32),
            pltpu.VMEM((2, BBLK, SEQ), jnp.int32),
            [pltpu.SemaphoreType.DMA] * NBUF,
            [pltpu.SemaphoreType.DMA] * NBUF,
        ],
        compiler_params=pltpu.CompilerParams(needs_layout_passes=False,
                                             use_tc_tiling_on_sc=False),
    )
    return kern(table, q4, d4)


def kernel(query_input, document_input, table):
    q4 = query_input.astype(jnp.int32).reshape(NW, BBLK, SEQ)
    d4 = document_input.astype(jnp.int32).reshape(NW, BBLK, SEQ)
    out6 = _run(table, q4, d4)
    return out6.transpose(0, 3, 5, 2, 4, 1).reshape(2, BATCH, EMBED_DIM, SEQ)
